# Initial kernel scaffold; baseline (speedup 1.0000x reference)
import jax
import jax.numpy as jnp
from jax.experimental import pallas as pl


def kernel(x, table):
    raise NotImplementedError("write your pallas kernel here")


# speedup vs baseline: 1.6956x; 1.6956x over previous
import functools

import jax
import jax.numpy as jnp
from jax import lax
from jax.experimental import pallas as pl
from jax.experimental.pallas import tpu as pltpu
from jax.experimental.pallas import tpu_sc as plsc

VOCAB_ROWS = 1_000_000
EMB_DIM = 64
NUM_CORES = 2
NUM_SUBCORES = 16
NUM_WORKERS = NUM_CORES * NUM_SUBCORES
CHUNK = 128


@functools.partial(jax.jit, static_argnames=("total_b",))
def _sc_gather(table, idx_flat, total_b):
  b_per_w = total_b // NUM_WORKERS
  n_chunks = b_per_w // CHUNK
  mesh = plsc.VectorSubcoreMesh(core_axis_name="c", subcore_axis_name="s")

  @functools.partial(
      pl.kernel,
      out_type=jax.ShapeDtypeStruct((total_b, EMB_DIM), jnp.float32),
      mesh=mesh,
      scratch_types=[
          pltpu.VMEM((b_per_w,), jnp.int32),
          pltpu.VMEM((CHUNK, EMB_DIM), jnp.float32),
          pltpu.SemaphoreType.DMA,
      ],
      compiler_params=pltpu.CompilerParams(use_tc_tiling_on_sc=False),
  )
  def body(table_hbm, idx_hbm, out_hbm, idx_v, rows_v, sem):
    wid = lax.axis_index("s") * NUM_CORES + lax.axis_index("c")
    base = wid * b_per_w
    pltpu.sync_copy(idx_hbm.at[pl.ds(base, b_per_w)], idx_v)

    def step(j, carry):
      off = j * CHUNK
      pltpu.async_copy(
          table_hbm.at[idx_v.at[pl.ds(off, CHUNK)]], rows_v, sem
      ).wait()
      pltpu.sync_copy(rows_v, out_hbm.at[pl.ds(base + off, CHUNK)])
      return carry

    lax.fori_loop(0, n_chunks, step, 0)

  return body(table, idx_flat)


def kernel(x, table):
  b, s = x.shape
  idx_flat = x.reshape(-1).astype(jnp.int32)
  out = _sc_gather(table, idx_flat, b * s)
  return out.reshape(b, s, EMB_DIM)


# speedup vs baseline: 1.8777x; 1.1074x over previous
import functools

import jax
import jax.numpy as jnp
from jax import lax
from jax.experimental import pallas as pl
from jax.experimental.pallas import tpu as pltpu
from jax.experimental.pallas import tpu_sc as plsc

VOCAB_ROWS = 1_000_000
EMB_DIM = 64
NUM_CORES = 2
NUM_SUBCORES = 16
NUM_WORKERS = NUM_CORES * NUM_SUBCORES
CHUNK = 128
NBUF = 4


@functools.partial(jax.jit, static_argnames=("total_b",))
def _sc_gather(table, idx_flat, total_b):
  b_per_w = total_b // NUM_WORKERS
  n_chunks = b_per_w // CHUNK
  n_groups = n_chunks // NBUF
  mesh = plsc.VectorSubcoreMesh(core_axis_name="c", subcore_axis_name="s")

  @functools.partial(
      pl.kernel,
      out_type=jax.ShapeDtypeStruct((total_b, EMB_DIM), jnp.float32),
      mesh=mesh,
      scratch_types=[
          pltpu.VMEM((b_per_w,), jnp.int32),
          pltpu.VMEM((NBUF, CHUNK, EMB_DIM), jnp.float32),
          [pltpu.SemaphoreType.DMA] * NBUF,
          [pltpu.SemaphoreType.DMA] * NBUF,
      ],
      compiler_params=pltpu.CompilerParams(use_tc_tiling_on_sc=False),
  )
  def body(table_hbm, idx_hbm, out_hbm, idx_v, rows_v, gsems, wsems):
    wid = lax.axis_index("s") * NUM_CORES + lax.axis_index("c")
    base = wid * b_per_w
    pltpu.sync_copy(idx_hbm.at[pl.ds(base, b_per_w)], idx_v)

    def gather(j, b):
      pltpu.async_copy(
          table_hbm.at[idx_v.at[pl.ds(j * CHUNK, CHUNK)]],
          rows_v.at[b],
          gsems[b],
      )

    def wait_gather(b):
      pltpu.make_async_copy(
          table_hbm.at[idx_v.at[pl.ds(0, CHUNK)]], rows_v.at[b], gsems[b]
      ).wait()

    def write(j, b):
      pltpu.async_copy(
          rows_v.at[b], out_hbm.at[pl.ds(base + j * CHUNK, CHUNK)], wsems[b]
      )

    def wait_write(b):
      pltpu.make_async_copy(
          rows_v.at[b], out_hbm.at[pl.ds(base, CHUNK)], wsems[b]
      ).wait()

    for b in range(NBUF - 1):
      gather(b, b)

    for b in range(NBUF):
      if b == 0:
        gather(NBUF - 1, NBUF - 1)
      else:
        wait_write((b - 1) % NBUF)
        gather(b + NBUF - 1, (b - 1) % NBUF)
      wait_gather(b)
      write(b, b)

    def group(g, carry):
      for b in range(NBUF):
        j = g * NBUF + b
        wait_write((b - 1) % NBUF)
        gather(j + NBUF - 1, (b - 1) % NBUF)
        wait_gather(b)
        write(j, b)
      return carry

    lax.fori_loop(1, n_groups - 1, group, 0)

    for b in range(NBUF):
      j = (n_groups - 1) * NBUF + b
      if b == 0:
        wait_write((b - 1) % NBUF)
        gather(j + NBUF - 1, (b - 1) % NBUF)
      wait_gather(b)
      write(j, b)

    for b in range(NBUF):
      wait_write(b)

  return body(table, idx_flat)


def kernel(x, table):
  b, s = x.shape
  idx_flat = x.reshape(-1).astype(jnp.int32)
  out = _sc_gather(table, idx_flat, b * s)
  return out.reshape(b, s, EMB_DIM)
